# baseline (speedup 1.0000x reference)
import functools

import jax
import jax.numpy as jnp
from jax.experimental import pallas as pl

B, S, N_CLASSES, D = 1024, 50, 1000, 128
M = B * S
BM = 1024


def _matmul_body(x_ref, w_ref, o_ref):
    o_ref[...] = jnp.dot(
        x_ref[...].astype(jnp.bfloat16),
        w_ref[...].astype(jnp.bfloat16),
        preferred_element_type=jnp.float32,
    )


@functools.partial(jax.jit, static_argnames=())
def kernel(x_multi_hot, embedding_weight):
    x2 = x_multi_hot.reshape(M, N_CLASSES)
    out = pl.pallas_call(
        _matmul_body,
        grid=(M // BM,),
        in_specs=[
            pl.BlockSpec((BM, N_CLASSES), lambda i: (i, 0)),
            pl.BlockSpec((N_CLASSES, D), lambda i: (0, 0)),
        ],
        out_specs=pl.BlockSpec((BM, D), lambda i: (i, 0)),
        out_shape=jax.ShapeDtypeStruct((M, D), jnp.float32),
    )(x2, embedding_weight)
    return out.reshape(B, S, D)


# speedup vs baseline: 1.4499x; 1.4499x over previous
import jax
import jax.numpy as jnp
from jax.experimental import pallas as pl

B, S, N_CLASSES, D = 1024, 50, 1000, 128
BB = 32


def _matmul_body(x_ref, w_ref, o_ref):
    x2 = x_ref[...].reshape(BB * S, N_CLASSES)
    acc = jnp.dot(
        x2.astype(jnp.bfloat16),
        w_ref[...].astype(jnp.bfloat16),
        preferred_element_type=jnp.float32,
    )
    o_ref[...] = acc.reshape(BB, S, D)


def kernel(x_multi_hot, embedding_weight):
    return pl.pallas_call(
        _matmul_body,
        grid=(B // BB,),
        in_specs=[
            pl.BlockSpec((BB, S, N_CLASSES), lambda i: (i, 0, 0)),
            pl.BlockSpec((N_CLASSES, D), lambda i: (0, 0)),
        ],
        out_specs=pl.BlockSpec((BB, S, D), lambda i: (i, 0, 0)),
        out_shape=jax.ShapeDtypeStruct((B, S, D), jnp.float32),
    )(x_multi_hot, embedding_weight)


# speedup vs baseline: 5.5041x; 3.7962x over previous
import jax
import jax.numpy as jnp
from jax.experimental import pallas as pl

B, S, N_CLASSES, D = 1024, 50, 1000, 128


def _matmul_body(x_ref, w_ref, o_ref):
    xs = x_ref[0]
    acc = jax.lax.dot_general(
        xs.astype(jnp.bfloat16),
        w_ref[...].astype(jnp.bfloat16),
        (((0,), (0,)), ((), ())),
        preferred_element_type=jnp.float32,
    )
    o_ref[0] = acc


def kernel(x_multi_hot, embedding_weight):
    x_t = jnp.transpose(x_multi_hot, (1, 2, 0))
    out_t = pl.pallas_call(
        _matmul_body,
        grid=(S,),
        in_specs=[
            pl.BlockSpec((1, N_CLASSES, B), lambda i: (i, 0, 0)),
            pl.BlockSpec((N_CLASSES, D), lambda i: (0, 0)),
        ],
        out_specs=pl.BlockSpec((1, B, D), lambda i: (i, 0, 0)),
        out_shape=jax.ShapeDtypeStruct((S, B, D), jnp.float32),
    )(x_t, embedding_weight)
    return jnp.transpose(out_t, (1, 0, 2))
